# baseline (speedup 1.0000x reference)
import functools

import jax
import jax.numpy as jnp
from jax import lax
from jax.experimental import pallas as pl
from jax.experimental.pallas import tpu as pltpu
from jax.experimental.pallas import tpu_sc as plsc

N = 200000
H = 256
W = 256
T = 16
VOX = T * H * W

LANES = 128
N_PAD = 200704
NR = N_PAD // LANES
BR = 32
K = 27

E = K * N_PAD
ER = E // LANES

SC_TILES = 16
PT = E // SC_TILES
CH = 12096
NCH = PT // CH
SLICE = VOX // SC_TILES
BOUNCE = 16384
HOPS = SLICE // BOUNCE

_OFFS = [(dx, dy, dt) for dx in (-1, 0, 1) for dy in (-1, 0, 1)
         for dt in (-1, 0, 1)]


def _emit_body(xyz_ref, cho_ref, fea_ref, opa_ref, idx_ref, val_ref):
    j = pl.program_id(0)
    x3 = jnp.tanh(xyz_ref[0])
    y3 = jnp.tanh(xyz_ref[1])
    z3 = jnp.tanh(xyz_ref[2])
    px = (x3 + 1.0) * (0.5 * (W - 1))
    py = (y3 + 1.0) * (0.5 * (H - 1))
    pt = (z3 + 1.0) * (0.5 * (T - 1))
    cx = jnp.floor(px + 0.5)
    cy = jnp.floor(py + 0.5)
    ct = jnp.floor(pt + 0.5)

    c0 = cho_ref[0] + 0.5
    c1 = cho_ref[1]
    c2 = cho_ref[2] + 0.5
    c3 = cho_ref[3] + 0.5
    c4 = cho_ref[4]
    c5 = cho_ref[5] + 0.5
    eps = 1e-4
    v00 = c0 * c0 + eps
    v01 = c0 * c1
    v02 = c0 * c3
    v11 = c1 * c1 + c2 * c2 + eps
    v12 = c1 * c3 + c2 * c4
    v22 = c3 * c3 + c4 * c4 + c5 * c5 + eps
    m00 = v11 * v22 - v12 * v12
    m01 = v02 * v12 - v01 * v22
    m02 = v01 * v12 - v02 * v11
    det = v00 * m00 + v01 * m01 + v02 * m02
    rdet = 1.0 / det
    q00 = m00 * rdet
    q01 = m01 * rdet
    q02 = m02 * rdet
    q11 = (v00 * v22 - v02 * v02) * rdet
    q12 = (v01 * v02 - v00 * v12) * rdet
    q22 = (v00 * v11 - v01 * v01) * rdet

    alpha = jax.nn.sigmoid(opa_ref[0])
    colr = jax.nn.sigmoid(fea_ref[0])
    colg = jax.nn.sigmoid(fea_ref[1])
    colb = jax.nn.sigmoid(fea_ref[2])

    rows = jax.lax.broadcasted_iota(jnp.int32, (BR, LANES), 0)
    cols = jax.lax.broadcasted_iota(jnp.int32, (BR, LANES), 1)
    pos = (j * BR + rows) * LANES + cols
    valid = pos < N
    a_eff = jnp.where(valid, alpha, 0.0)

    for k, (dxo, dyo, dto) in enumerate(_OFFS):
        vx = jnp.clip(cx + dxo, 0.0, W - 1.0)
        vy = jnp.clip(cy + dyo, 0.0, H - 1.0)
        vt = jnp.clip(ct + dto, 0.0, T - 1.0)
        dxv = vx - px
        dyv = vy - py
        dtv = vt - pt
        quad = (q00 * dxv * dxv + q11 * dyv * dyv + q22 * dtv * dtv
                + 2.0 * (q01 * dxv * dyv + q02 * dxv * dtv
                         + q12 * dyv * dtv))
        w = jnp.exp(-0.5 * quad)
        contrib = a_eff * w
        ix = vx.astype(jnp.int32)
        iy = vy.astype(jnp.int32)
        it = vt.astype(jnp.int32)
        s = iy * (W * T) + ix * T + it
        idx_ref[k] = jnp.where(valid, s, pos)
        val_ref[k] = contrib * colr
        val_ref[K + k] = contrib * colg
        val_ref[2 * K + k] = contrib * colb
        val_ref[3 * K + k] = contrib


def _emit(xyz_t, cho_t, fea_t, opa_t):
    grid = NR // BR
    return pl.pallas_call(
        _emit_body,
        grid=(grid,),
        in_specs=[
            pl.BlockSpec((3, BR, LANES), lambda j: (0, j, 0)),
            pl.BlockSpec((6, BR, LANES), lambda j: (0, j, 0)),
            pl.BlockSpec((3, BR, LANES), lambda j: (0, j, 0)),
            pl.BlockSpec((1, BR, LANES), lambda j: (0, j, 0)),
        ],
        out_specs=[
            pl.BlockSpec((K, BR, LANES), lambda j: (0, j, 0)),
            pl.BlockSpec((4 * K, BR, LANES), lambda j: (0, j, 0)),
        ],
        out_shape=[
            jax.ShapeDtypeStruct((K, NR, LANES), jnp.int32),
            jax.ShapeDtypeStruct((4 * K, NR, LANES), jnp.float32),
        ],
    )(xyz_t, cho_t, fea_t, opa_t)


def _scatter_body(idx_hbm, val_hbm, acc_hbm, idx0, val0, idx1, val1, bounce,
                  shared, sem0, sem1):
    cid = lax.axis_index("c")
    sid = lax.axis_index("s")
    ent0 = sid * PT
    vox0 = sid * SLICE

    bufs = ((idx0, val0, sem0), (idx1, val1, sem1))

    for half in range(2):
        ch = 2 * cid + half

        def start(g, ib, vb, sem):
            e = pl.multiple_of(ent0 + g * CH, 64)
            pltpu.async_copy(idx_hbm.at[pl.ds(e, CH)], ib, sem)
            ve = pl.multiple_of(ch * E + e, 64)
            pltpu.async_copy(val_hbm.at[pl.ds(ve, CH)], vb, sem)

        def drain(ib, vb, sem):
            pltpu.make_async_copy(idx_hbm.at[pl.ds(0, CH)], ib, sem).wait()
            pltpu.make_async_copy(val_hbm.at[pl.ds(0, CH)], vb, sem).wait()

        def scatter(ib, vb):
            pltpu.sync_copy(vb, shared.at[ib], add=True)

        start(0, *bufs[0])

        def zero_body(i, _):
            bounce[pl.ds(i * 16, 16)] = jnp.zeros((16,), jnp.float32)
            return 0
        lax.fori_loop(0, BOUNCE // 16, zero_body, 0)
        for h in range(HOPS):
            pltpu.sync_copy(bounce,
                            shared.at[pl.ds(vox0 + h * BOUNCE, BOUNCE)])
        plsc.subcore_barrier()

        def pair_body(i, _):
            g = 2 * i
            start(g + 1, *bufs[1])
            drain(*bufs[0])
            scatter(bufs[0][0], bufs[0][1])
            start(g + 2, *bufs[0])
            drain(*bufs[1])
            scatter(bufs[1][0], bufs[1][1])
            return 0
        lax.fori_loop(0, (NCH - 1) // 2, pair_body, 0)
        if NCH % 2 == 0:
            start(NCH - 1, *bufs[1])
            drain(*bufs[0])
            scatter(bufs[0][0], bufs[0][1])
            drain(*bufs[1])
            scatter(bufs[1][0], bufs[1][1])
        else:
            drain(*bufs[0])
            scatter(bufs[0][0], bufs[0][1])
        plsc.subcore_barrier()

        for h in range(HOPS):
            pltpu.sync_copy(shared.at[pl.ds(vox0 + h * BOUNCE, BOUNCE)],
                            bounce)
            a = pl.multiple_of(ch * VOX + vox0 + h * BOUNCE, 64)
            pltpu.sync_copy(bounce, acc_hbm.at[pl.ds(a, BOUNCE)])


def _scatter(idx, vals):
    mesh = plsc.VectorSubcoreMesh(core_axis_name="c", subcore_axis_name="s")
    f = pl.kernel(
        _scatter_body,
        out_type=jax.ShapeDtypeStruct((4 * VOX,), jnp.float32),
        mesh=mesh,
        scratch_types=[
            pltpu.VMEM((CH,), jnp.int32),
            pltpu.VMEM((CH,), jnp.float32),
            pltpu.VMEM((CH,), jnp.int32),
            pltpu.VMEM((CH,), jnp.float32),
            pltpu.VMEM((BOUNCE,), jnp.float32),
            pltpu.VMEM_SHARED((VOX,), jnp.float32),
            pltpu.SemaphoreType.DMA,
            pltpu.SemaphoreType.DMA,
        ],
    )
    return f(idx, vals)


BY = 32


def _blend_body(acc_ref, out_ref):
    a = acc_ref[3]
    bg = jnp.clip(1.0 - a, 0.0, 1.0)
    for c in range(3):
        o = jnp.clip(acc_ref[c] + bg, 0.0, 1.0)
        out_ref[0, c] = o.reshape(BY, W, T)


def _blend(acc):
    return pl.pallas_call(
        _blend_body,
        grid=(H // BY,),
        in_specs=[pl.BlockSpec((4, BY, W * T), lambda j: (0, j, 0))],
        out_specs=pl.BlockSpec((1, 3, BY, W, T), lambda j: (0, 0, j, 0, 0)),
        out_shape=jax.ShapeDtypeStruct((1, 3, H, W, T), jnp.float32),
    )(acc.reshape(4, H, W * T))


def kernel(xyz_3d, cholesky_3d, features_dc, opacity):
    pad = N_PAD - N
    xyz_t = jnp.pad(xyz_3d, ((0, pad), (0, 0))).T.reshape(3, NR, LANES)
    cho_t = jnp.pad(cholesky_3d, ((0, pad), (0, 0))).T.reshape(6, NR, LANES)
    fea_t = jnp.pad(features_dc, ((0, pad), (0, 0))).T.reshape(3, NR, LANES)
    opa_t = jnp.pad(opacity, ((0, pad), (0, 0))).T.reshape(1, NR, LANES)

    idx, vals = _emit(xyz_t, cho_t, fea_t, opa_t)
    idx = idx.reshape(E)
    vals = vals.reshape(4 * E)

    acc = _scatter(idx, vals)
    return _blend(acc)


# speedup vs baseline: 1.0596x; 1.0596x over previous
import functools

import jax
import jax.numpy as jnp
from jax import lax
from jax.experimental import pallas as pl
from jax.experimental.pallas import tpu as pltpu
from jax.experimental.pallas import tpu_sc as plsc

N = 200000
H = 256
W = 256
T = 16
VOX = T * H * W

LANES = 128
N_PAD = 200704
NR = N_PAD // LANES
BR = 32
K = 27

E = K * N_PAD
ER = E // LANES

SC_TILES = 16
PT = E // SC_TILES
CH = 12096
NCH = PT // CH
SLICE = VOX // SC_TILES
BOUNCE = 16384
HOPS = SLICE // BOUNCE

_OFFS = [(dx, dy, dt) for dx in (-1, 0, 1) for dy in (-1, 0, 1)
         for dt in (-1, 0, 1)]


def _emit_body(xyz_ref, cho_ref, fea_ref, opa_ref, idx_ref, val_ref):
    j = pl.program_id(0)
    x3 = jnp.tanh(xyz_ref[0])
    y3 = jnp.tanh(xyz_ref[1])
    z3 = jnp.tanh(xyz_ref[2])
    px = (x3 + 1.0) * (0.5 * (W - 1))
    py = (y3 + 1.0) * (0.5 * (H - 1))
    pt = (z3 + 1.0) * (0.5 * (T - 1))
    cx = jnp.floor(px + 0.5)
    cy = jnp.floor(py + 0.5)
    ct = jnp.floor(pt + 0.5)

    c0 = cho_ref[0] + 0.5
    c1 = cho_ref[1]
    c2 = cho_ref[2] + 0.5
    c3 = cho_ref[3] + 0.5
    c4 = cho_ref[4]
    c5 = cho_ref[5] + 0.5
    eps = 1e-4
    v00 = c0 * c0 + eps
    v01 = c0 * c1
    v02 = c0 * c3
    v11 = c1 * c1 + c2 * c2 + eps
    v12 = c1 * c3 + c2 * c4
    v22 = c3 * c3 + c4 * c4 + c5 * c5 + eps
    m00 = v11 * v22 - v12 * v12
    m01 = v02 * v12 - v01 * v22
    m02 = v01 * v12 - v02 * v11
    det = v00 * m00 + v01 * m01 + v02 * m02
    rdet = 1.0 / det
    q00 = m00 * rdet
    q01 = m01 * rdet
    q02 = m02 * rdet
    q11 = (v00 * v22 - v02 * v02) * rdet
    q12 = (v01 * v02 - v00 * v12) * rdet
    q22 = (v00 * v11 - v01 * v01) * rdet

    alpha = jax.nn.sigmoid(opa_ref[0])
    colr = jax.nn.sigmoid(fea_ref[0])
    colg = jax.nn.sigmoid(fea_ref[1])
    colb = jax.nn.sigmoid(fea_ref[2])

    rows = jax.lax.broadcasted_iota(jnp.int32, (BR, LANES), 0)
    cols = jax.lax.broadcasted_iota(jnp.int32, (BR, LANES), 1)
    pos = (j * BR + rows) * LANES + cols
    valid = pos < N
    a_eff = jnp.where(valid, alpha, 0.0)

    for k, (dxo, dyo, dto) in enumerate(_OFFS):
        vx = jnp.clip(cx + dxo, 0.0, W - 1.0)
        vy = jnp.clip(cy + dyo, 0.0, H - 1.0)
        vt = jnp.clip(ct + dto, 0.0, T - 1.0)
        dxv = vx - px
        dyv = vy - py
        dtv = vt - pt
        quad = (q00 * dxv * dxv + q11 * dyv * dyv + q22 * dtv * dtv
                + 2.0 * (q01 * dxv * dyv + q02 * dxv * dtv
                         + q12 * dyv * dtv))
        w = jnp.exp(-0.5 * quad)
        contrib = a_eff * w
        ix = vx.astype(jnp.int32)
        iy = vy.astype(jnp.int32)
        it = vt.astype(jnp.int32)
        s = iy * (W * T) + ix * T + it
        idx_ref[k] = jnp.where(valid, s, pos)
        val_ref[k] = contrib * colr
        val_ref[K + k] = contrib * colg
        val_ref[2 * K + k] = contrib * colb
        val_ref[3 * K + k] = contrib


def _emit(xyz_t, cho_t, fea_t, opa_t):
    grid = NR // BR
    return pl.pallas_call(
        _emit_body,
        grid=(grid,),
        in_specs=[
            pl.BlockSpec((3, BR, LANES), lambda j: (0, j, 0)),
            pl.BlockSpec((6, BR, LANES), lambda j: (0, j, 0)),
            pl.BlockSpec((3, BR, LANES), lambda j: (0, j, 0)),
            pl.BlockSpec((1, BR, LANES), lambda j: (0, j, 0)),
        ],
        out_specs=[
            pl.BlockSpec((K, BR, LANES), lambda j: (0, j, 0)),
            pl.BlockSpec((4 * K, BR, LANES), lambda j: (0, j, 0)),
        ],
        out_shape=[
            jax.ShapeDtypeStruct((K, NR, LANES), jnp.int32),
            jax.ShapeDtypeStruct((4 * K, NR, LANES), jnp.float32),
        ],
    )(xyz_t, cho_t, fea_t, opa_t)


def _scatter_body(idx_hbm, val_hbm, acc_hbm, idx0, val0, idx1, val1, bounce,
                  shared, sem0, sem1):
    cid = lax.axis_index("c")
    sid = lax.axis_index("s")
    ent0 = sid * PT
    vox0 = sid * SLICE

    bufs = ((idx0, val0, sem0), (idx1, val1, sem1))

    for half in range(2):
        ch = 2 * cid + half

        def start(g, ib, vb, sem):
            e = pl.multiple_of(ent0 + g * CH, 64)
            pltpu.async_copy(idx_hbm.at[pl.ds(e, CH)], ib, sem)
            ve = pl.multiple_of(ch * E + e, 64)
            pltpu.async_copy(val_hbm.at[pl.ds(ve, CH)], vb, sem)

        def drain(ib, vb, sem):
            pltpu.make_async_copy(idx_hbm.at[pl.ds(0, CH)], ib, sem).wait()
            pltpu.make_async_copy(val_hbm.at[pl.ds(0, CH)], vb, sem).wait()

        def scatter(ib, vb):
            pltpu.sync_copy(vb, shared.at[ib], add=True)

        start(0, *bufs[0])

        def zero_body(i, _):
            bounce[pl.ds(i * 16, 16)] = jnp.zeros((16,), jnp.float32)
            return 0
        lax.fori_loop(0, BOUNCE // 16, zero_body, 0)
        for h in range(HOPS):
            pltpu.sync_copy(bounce,
                            shared.at[pl.ds(vox0 + h * BOUNCE, BOUNCE)])
        plsc.subcore_barrier()

        def pair_body(i, _):
            g = 2 * i
            start(g + 1, *bufs[1])
            drain(*bufs[0])
            scatter(bufs[0][0], bufs[0][1])
            start(g + 2, *bufs[0])
            drain(*bufs[1])
            scatter(bufs[1][0], bufs[1][1])
            return 0
        lax.fori_loop(0, (NCH - 1) // 2, pair_body, 0)
        if NCH % 2 == 0:
            start(NCH - 1, *bufs[1])
            drain(*bufs[0])
            scatter(bufs[0][0], bufs[0][1])
            drain(*bufs[1])
            scatter(bufs[1][0], bufs[1][1])
        else:
            drain(*bufs[0])
            scatter(bufs[0][0], bufs[0][1])
        plsc.subcore_barrier()

        for h in range(HOPS):
            pltpu.sync_copy(shared.at[pl.ds(vox0 + h * BOUNCE, BOUNCE)],
                            bounce)
            a = pl.multiple_of(ch * VOX + vox0 + h * BOUNCE, 64)
            pltpu.sync_copy(bounce, acc_hbm.at[pl.ds(a, BOUNCE)])


def _scatter(idx, vals):
    mesh = plsc.VectorSubcoreMesh(core_axis_name="c", subcore_axis_name="s")
    f = pl.kernel(
        _scatter_body,
        out_type=jax.ShapeDtypeStruct((4 * VOX,), jnp.float32),
        mesh=mesh,
        scratch_types=[
            pltpu.VMEM((CH,), jnp.int32),
            pltpu.VMEM((CH,), jnp.float32),
            pltpu.VMEM((CH,), jnp.int32),
            pltpu.VMEM((CH,), jnp.float32),
            pltpu.VMEM((BOUNCE,), jnp.float32),
            pltpu.VMEM_SHARED((VOX,), jnp.float32),
            pltpu.SemaphoreType.DMA,
            pltpu.SemaphoreType.DMA,
        ],
    )
    return f(idx, vals)


def _blend_body(acc_ref, out_ref):
    a = acc_ref[3]
    bg = jnp.clip(1.0 - a, 0.0, 1.0)
    for c in range(3):
        out_ref[c] = jnp.clip(acc_ref[c] + bg, 0.0, 1.0)


def _blend(acc):
    rows = VOX // LANES
    br = 1024
    return pl.pallas_call(
        _blend_body,
        grid=(rows // br,),
        in_specs=[pl.BlockSpec((4, br, LANES), lambda j: (0, j, 0))],
        out_specs=pl.BlockSpec((3, br, LANES), lambda j: (0, j, 0)),
        out_shape=jax.ShapeDtypeStruct((3, rows, LANES), jnp.float32),
    )(acc.reshape(4, rows, LANES))


def kernel(xyz_3d, cholesky_3d, features_dc, opacity):
    pad = N_PAD - N
    xyz_t = jnp.pad(xyz_3d, ((0, pad), (0, 0))).T.reshape(3, NR, LANES)
    cho_t = jnp.pad(cholesky_3d, ((0, pad), (0, 0))).T.reshape(6, NR, LANES)
    fea_t = jnp.pad(features_dc, ((0, pad), (0, 0))).T.reshape(3, NR, LANES)
    opa_t = jnp.pad(opacity, ((0, pad), (0, 0))).T.reshape(1, NR, LANES)

    idx, vals = _emit(xyz_t, cho_t, fea_t, opa_t)
    idx = idx.reshape(E)
    vals = vals.reshape(4 * E)

    acc = _scatter(idx, vals)
    out = _blend(acc)
    return out.reshape(1, 3, H, W, T)
